# baseline (speedup 1.0000x reference)
import dataclasses
import functools

import jax
import jax.numpy as jnp
from jax import lax
from jax.experimental import pallas as pl
from jax.experimental.pallas import tpu as pltpu
from jax.experimental.pallas import tpu_sc as plsc

_BATCH = 16384
_D = 64
_DP = 128
_NC = 2
_NS = 16
_LANES = 16
_NW = _NC * _NS
_BPW = _BATCH // _NW
_CHUNK = 128
_NCHUNK = _BPW // _CHUNK


def _dmf_body(uid_hbm, iid_hbm, ut_hbm, it_hbm, out_hbm,
              uidx, iidx, urows, irows, outv, sem0, sem1):
    wid = lax.axis_index("s") * _NC + lax.axis_index("c")
    base = wid * _BPW
    sems = (sem0, sem1)

    for j in range(_NCHUNK):
        pltpu.sync_copy(uid_hbm.at[pl.ds(base + j * _CHUNK, _CHUNK)], uidx.at[j])
        pltpu.sync_copy(iid_hbm.at[pl.ds(base + j * _CHUNK, _CHUNK)], iidx.at[j])
    for j in range(_NCHUNK):
        for v in range(_CHUNK // _LANES):
            sl = pl.ds(v * _LANES, _LANES)
            uidx[_NCHUNK + j, sl] = lax.shift_right_logical(uidx[j, sl], 1)
            iidx[_NCHUNK + j, sl] = lax.shift_right_logical(iidx[j, sl], 1)

    lane = lax.iota(jnp.int32, _LANES)

    def fire(j):
        s = j % 2
        return (
            pltpu.async_copy(ut_hbm.at[uidx.at[_NCHUNK + j]], urows.at[s], sems[s]),
            pltpu.async_copy(it_hbm.at[iidx.at[_NCHUNK + j]], irows.at[s], sems[s]),
        )

    pending = {0: fire(0)}
    for j in range(_NCHUNK):
        if j + 1 < _NCHUNK:
            pending[j + 1] = fire(j + 1)
        for c in pending.pop(j):
            c.wait()
        s = j % 2
        u2d, i2d = urows.at[s], irows.at[s]

        for g in range(_CHUNK // _LANES):
            row = g * _LANES + lane
            sl = pl.ds(g * _LANES, _LANES)
            ucol0 = (uidx[j, sl] & 1) * _D
            icol0 = (iidx[j, sl] & 1) * _D

            def body(d, acc):
                u = plsc.load_gather(u2d, [row, ucol0 + d])
                v = plsc.load_gather(i2d, [row, icol0 + d])
                return acc + u * v

            acc = lax.fori_loop(0, _D, body, jnp.zeros((_LANES,), jnp.float32))
            outv[pl.ds(j * _CHUNK + g * _LANES, _LANES)] = acc

    pltpu.sync_copy(outv, out_hbm.at[pl.ds(base, _BPW)])


def _compiler_params():
    cp = pltpu.CompilerParams(disable_bounds_checks=True)
    if "needs_layout_passes" in pltpu.CompilerParams.__dataclass_fields__:
        cp = dataclasses.replace(cp, needs_layout_passes=False)
    return cp


@jax.jit
def _dmf(user_ids, item_ids, user_table, item_table):
    k = pl.kernel(
        _dmf_body,
        out_type=jax.ShapeDtypeStruct((_BATCH,), jnp.float32),
        mesh=plsc.VectorSubcoreMesh(core_axis_name="c", subcore_axis_name="s"),
        compiler_params=_compiler_params(),
        scratch_types=[
            pltpu.VMEM((2 * _NCHUNK, _CHUNK), jnp.int32),
            pltpu.VMEM((2 * _NCHUNK, _CHUNK), jnp.int32),
            pltpu.VMEM((2, _CHUNK, _DP), jnp.float32),
            pltpu.VMEM((2, _CHUNK, _DP), jnp.float32),
            pltpu.VMEM((_BPW,), jnp.float32),
            pltpu.SemaphoreType.DMA,
            pltpu.SemaphoreType.DMA,
        ],
    )
    ut2 = user_table.reshape(user_table.shape[0] // 2, _DP)
    it2 = item_table.reshape(item_table.shape[0] // 2, _DP)
    return k(user_ids, item_ids, ut2, it2)


def kernel(user_ids, item_ids, user_table, item_table):
    return _dmf(user_ids, item_ids, user_table, item_table)
